# baseline (speedup 1.0000x reference)
import functools

import jax
import jax.numpy as jnp
from jax.experimental import pallas as pl

N = 10000
E = 160000
C = 20000
B = 64
H = 32
NF = 128
EF = 16
EH = 64
STEPS = 3

MSG_TILE = 1000


def _bn(x, g, b):
    m = jnp.mean(x, axis=0)
    v = jnp.var(x, axis=0)
    return g * (x - m) / jnp.sqrt(v + 1e-5) + b


def _msg_body(a_ref, e4_ref, w4t_ref, br_ref, out_ref):
    wt = jnp.dot(e4_ref[...], w4t_ref[...], preferred_element_type=jnp.float32)
    a = a_ref[...]
    msg = jnp.dot(a, br_ref[...], preferred_element_type=jnp.float32)
    for i in range(H):
        msg += a[:, i:i + 1] * wt[:, i * H:(i + 1) * H]
    out_ref[...] = msg


@functools.partial(jax.jit, static_argnames=("tile",))
def _msg_matmul(a_src, e4, w4t, br, tile=MSG_TILE):
    return pl.pallas_call(
        _msg_body,
        grid=(E // tile,),
        in_specs=[
            pl.BlockSpec((tile, H), lambda i: (i, 0)),
            pl.BlockSpec((tile, EH), lambda i: (i, 0)),
            pl.BlockSpec((EH, H * H), lambda i: (0, 0)),
            pl.BlockSpec((H, H), lambda i: (0, 0)),
        ],
        out_specs=pl.BlockSpec((tile, H), lambda i: (i, 0)),
        out_shape=jax.ShapeDtypeStruct((E, H), jnp.float32),
    )(a_src, e4, w4t, br)


def kernel(node, edge, edge_index, node_batch_index, coupling_index, coupling_type, coupling_type_back, coupling_value, coupling_batch_index, params):
    p = params
    ei = edge_index.T
    src = ei[0]
    dst = ei[1]
    x = jax.nn.relu(_bn(node, p['emb_bn1_g'], p['emb_bn1_b']) @ p['emb_W1'].T + p['emb_b1'])
    x = _bn(x, p['emb_bn2_g'], p['emb_bn2_b']) @ p['emb_W2'].T
    x = jax.nn.relu(x)
    h = x
    acc = x
    e = jax.nn.relu(_bn(edge, p['en_bn1_g'], p['en_bn1_b']) @ p['en_W1'].T + p['en_b1'])
    e = jax.nn.relu(_bn(e, p['en_bn2_g'], p['en_bn2_b']) @ p['en_W2'].T + p['en_b2'])
    e = jax.nn.relu(_bn(e, p['en_bn3_g'], p['en_bn3_b']) @ p['en_W3'].T + p['en_b3'])
    e4 = _bn(e, p['en_bn4_g'], p['en_bn4_b'])
    counts = jnp.maximum(jax.ops.segment_sum(jnp.ones(E, jnp.float32), dst, num_segments=N), 1.0)[:, None]
    w4t = p['en_W4'].T
    br = p['en_b4'].reshape(H, H)
    for _ in range(STEPS):
        a_src = acc[src]
        msg = _msg_matmul(a_src, e4, w4t, br)
        agg = jax.ops.segment_sum(msg, dst, num_segments=N) / counts
        m = jax.nn.relu(agg + acc @ p['conv_root'].T + p['conv_bias'])
        gi = m @ p['gru_W_ih'].T + p['gru_b_ih']
        gh = h @ p['gru_W_hh'].T + p['gru_b_hh']
        i_r, i_z, i_n = jnp.split(gi, 3, axis=1)
        h_r, h_z, h_n = jnp.split(gh, 3, axis=1)
        r = jax.nn.sigmoid(i_r + h_r)
        z = jax.nn.sigmoid(i_z + h_z)
        n = jnp.tanh(i_n + r * h_n)
        acc = (1.0 - z) * n + z * h
        h = acc
    q_star = jnp.zeros((B, 2 * H), jnp.float32)
    hl = jnp.zeros((B, H), jnp.float32)
    cl = jnp.zeros((B, H), jnp.float32)
    for _ in range(STEPS):
        gates = q_star @ p['lstm_W_ih'].T + p['lstm_b_ih'] + hl @ p['lstm_W_hh'].T + p['lstm_b_hh']
        gi_, gf_, gg_, go_ = jnp.split(gates, 4, axis=1)
        cl = jax.nn.sigmoid(gf_) * cl + jax.nn.sigmoid(gi_) * jnp.tanh(gg_)
        hl = jax.nn.sigmoid(go_) * jnp.tanh(cl)
        eatt = jnp.sum(acc * hl[node_batch_index], axis=-1)
        emax = jax.ops.segment_max(eatt, node_batch_index, num_segments=B)
        a = jnp.exp(eatt - emax[node_batch_index])
        denom = jax.ops.segment_sum(a, node_batch_index, num_segments=B)
        a = a / (denom[node_batch_index] + 1e-16)
        r_ = jax.ops.segment_sum(a[:, None] * acc, node_batch_index, num_segments=B)
        q_star = jnp.concatenate([hl, r_], axis=1)
    pool = q_star[coupling_batch_index]
    nf = acc[coupling_index.reshape(-1)].reshape(C, -1)
    feats = jnp.concatenate([pool, nf, coupling_type.astype(jnp.float32)], axis=-1)
    zf = jax.nn.relu(_bn(feats, p['fc_bn1_g'], p['fc_bn1_b']) @ p['fc_W1'].T + p['fc_b1'])
    preds = _bn(zf, p['fc_bn2_g'], p['fc_bn2_b']) @ p['fc_W2'].T + p['fc_b2']
    pred = jnp.take_along_axis(preds, coupling_type_back[:, None], axis=1).reshape(-1)
    return pred


# speedup vs baseline: 1.1177x; 1.1177x over previous
import functools

import jax
import jax.numpy as jnp
from jax import lax
from jax.experimental import pallas as pl
from jax.experimental.pallas import tpu as pltpu
from jax.experimental.pallas import tpu_sc as plsc

N = 10000
E = 160000
C = 20000
B = 64
H = 32
NF = 128
EF = 16
EH = 64
STEPS = 3

MSG_TILE = 1000


def _bn(x, g, b):
    m = jnp.mean(x, axis=0)
    v = jnp.var(x, axis=0)
    return g * (x - m) / jnp.sqrt(v + 1e-5) + b


def _msg_body(a_ref, e4_ref, w4t_ref, br_ref, out_ref):
    wt = jnp.dot(e4_ref[...], w4t_ref[...], preferred_element_type=jnp.float32)
    a = a_ref[...]
    msg = jnp.dot(a, br_ref[...], preferred_element_type=jnp.float32)
    for i in range(H):
        msg += a[:, i:i + 1] * wt[:, i * H:(i + 1) * H]
    out_ref[...] = msg


@functools.partial(jax.jit, static_argnames=("tile",))
def _msg_matmul(a_src, e4, w4t, br, tile=MSG_TILE):
    return pl.pallas_call(
        _msg_body,
        grid=(E // tile,),
        in_specs=[
            pl.BlockSpec((tile, H), lambda i: (i, 0)),
            pl.BlockSpec((tile, EH), lambda i: (i, 0)),
            pl.BlockSpec((EH, H * H), lambda i: (0, 0)),
            pl.BlockSpec((H, H), lambda i: (0, 0)),
        ],
        out_specs=pl.BlockSpec((tile, H), lambda i: (i, 0)),
        out_shape=jax.ShapeDtypeStruct((E, H), jnp.float32),
    )(a_src, e4, w4t, br)


_SC_NC = 2
_SC_NS = 16
_CH = 128
_NCHUNK = E // _CH
_CPC = _NCHUNK // _SC_NC
_STRIPE = 624
_TAIL = N - _STRIPE * _SC_NS


@jax.jit
def _sc_scatter_add(msg3, dst_flat, zstripe):
    mesh = plsc.VectorSubcoreMesh(core_axis_name="c", subcore_axis_name="s")

    @functools.partial(
        pl.kernel, mesh=mesh,
        out_type=jax.ShapeDtypeStruct((_SC_NC, N, H), jnp.float32),
        scratch_types=[
            pltpu.VMEM((_CH,), jnp.int32),
            pltpu.VMEM((_CH, H), jnp.float32),
            pltpu.VMEM_SHARED((N, H), jnp.float32),
        ],
        compiler_params=pltpu.CompilerParams(use_tc_tiling_on_sc=False),
    )
    def k(msg_hbm, dst_hbm, z_hbm, out_hbm, idx_v, rows_v, agg_sh):
        c = lax.axis_index("c")
        s = lax.axis_index("s")
        pltpu.sync_copy(z_hbm, agg_sh.at[pl.ds(s * _STRIPE, _STRIPE)])

        @pl.when(s == _SC_NS - 1)
        def _():
            pltpu.sync_copy(z_hbm.at[pl.ds(0, _TAIL)],
                            agg_sh.at[pl.ds(_STRIPE * _SC_NS, _TAIL)])
        plsc.subcore_barrier()

        def body(j, carry):
            cid_local = j * _SC_NS + s

            @pl.when(cid_local < _CPC)
            def _():
                cid = c * _CPC + cid_local
                pltpu.sync_copy(dst_hbm.at[pl.ds(cid * _CH, _CH)], idx_v)
                pltpu.sync_copy(msg_hbm.at[cid], rows_v)
                pltpu.sync_copy(rows_v, agg_sh.at[idx_v], add=True)
            return carry

        lax.fori_loop(0, (_CPC + _SC_NS - 1) // _SC_NS, body, 0)
        plsc.subcore_barrier()
        pltpu.sync_copy(agg_sh.at[pl.ds(s * _STRIPE, _STRIPE)],
                        out_hbm.at[c].at[pl.ds(s * _STRIPE, _STRIPE)])

        @pl.when(s == _SC_NS - 1)
        def _():
            pltpu.sync_copy(agg_sh.at[pl.ds(_STRIPE * _SC_NS, _TAIL)],
                            out_hbm.at[c].at[pl.ds(_STRIPE * _SC_NS, _TAIL)])

    return k(msg3, dst_flat, zstripe)


def _segment_sum_sc(msg, dst_flat, zstripe):
    parts = _sc_scatter_add(msg.reshape(_NCHUNK, _CH, H), dst_flat, zstripe)
    return parts[0] + parts[1]


def kernel(node, edge, edge_index, node_batch_index, coupling_index, coupling_type, coupling_type_back, coupling_value, coupling_batch_index, params):
    p = params
    ei = edge_index.T
    src = ei[0]
    dst = ei[1]
    x = jax.nn.relu(_bn(node, p['emb_bn1_g'], p['emb_bn1_b']) @ p['emb_W1'].T + p['emb_b1'])
    x = _bn(x, p['emb_bn2_g'], p['emb_bn2_b']) @ p['emb_W2'].T
    x = jax.nn.relu(x)
    h = x
    acc = x
    e = jax.nn.relu(_bn(edge, p['en_bn1_g'], p['en_bn1_b']) @ p['en_W1'].T + p['en_b1'])
    e = jax.nn.relu(_bn(e, p['en_bn2_g'], p['en_bn2_b']) @ p['en_W2'].T + p['en_b2'])
    e = jax.nn.relu(_bn(e, p['en_bn3_g'], p['en_bn3_b']) @ p['en_W3'].T + p['en_b3'])
    e4 = _bn(e, p['en_bn4_g'], p['en_bn4_b'])
    counts = jnp.maximum(jax.ops.segment_sum(jnp.ones(E, jnp.float32), dst, num_segments=N), 1.0)[:, None]
    w4t = p['en_W4'].T
    br = p['en_b4'].reshape(H, H)
    dst_flat = dst.astype(jnp.int32)
    zstripe = jnp.zeros((_STRIPE, H), jnp.float32)
    for _ in range(STEPS):
        a_src = acc[src]
        msg = _msg_matmul(a_src, e4, w4t, br)
        agg = _segment_sum_sc(msg, dst_flat, zstripe) / counts
        m = jax.nn.relu(agg + acc @ p['conv_root'].T + p['conv_bias'])
        gi = m @ p['gru_W_ih'].T + p['gru_b_ih']
        gh = h @ p['gru_W_hh'].T + p['gru_b_hh']
        i_r, i_z, i_n = jnp.split(gi, 3, axis=1)
        h_r, h_z, h_n = jnp.split(gh, 3, axis=1)
        r = jax.nn.sigmoid(i_r + h_r)
        z = jax.nn.sigmoid(i_z + h_z)
        n = jnp.tanh(i_n + r * h_n)
        acc = (1.0 - z) * n + z * h
        h = acc
    q_star = jnp.zeros((B, 2 * H), jnp.float32)
    hl = jnp.zeros((B, H), jnp.float32)
    cl = jnp.zeros((B, H), jnp.float32)
    for _ in range(STEPS):
        gates = q_star @ p['lstm_W_ih'].T + p['lstm_b_ih'] + hl @ p['lstm_W_hh'].T + p['lstm_b_hh']
        gi_, gf_, gg_, go_ = jnp.split(gates, 4, axis=1)
        cl = jax.nn.sigmoid(gf_) * cl + jax.nn.sigmoid(gi_) * jnp.tanh(gg_)
        hl = jax.nn.sigmoid(go_) * jnp.tanh(cl)
        eatt = jnp.sum(acc * hl[node_batch_index], axis=-1)
        emax = jax.ops.segment_max(eatt, node_batch_index, num_segments=B)
        a = jnp.exp(eatt - emax[node_batch_index])
        denom = jax.ops.segment_sum(a, node_batch_index, num_segments=B)
        a = a / (denom[node_batch_index] + 1e-16)
        r_ = jax.ops.segment_sum(a[:, None] * acc, node_batch_index, num_segments=B)
        q_star = jnp.concatenate([hl, r_], axis=1)
    pool = q_star[coupling_batch_index]
    nf = acc[coupling_index.reshape(-1)].reshape(C, -1)
    feats = jnp.concatenate([pool, nf, coupling_type.astype(jnp.float32)], axis=-1)
    zf = jax.nn.relu(_bn(feats, p['fc_bn1_g'], p['fc_bn1_b']) @ p['fc_W1'].T + p['fc_b1'])
    preds = _bn(zf, p['fc_bn2_g'], p['fc_bn2_b']) @ p['fc_W2'].T + p['fc_b2']
    pred = jnp.take_along_axis(preds, coupling_type_back[:, None], axis=1).reshape(-1)
    return pred


# speedup vs baseline: 2.0521x; 1.8360x over previous
import functools

import jax
import jax.numpy as jnp
from jax import lax
from jax.experimental import pallas as pl
from jax.experimental.pallas import tpu as pltpu
from jax.experimental.pallas import tpu_sc as plsc

N = 10000
E = 160000
C = 20000
B = 64
H = 32
NF = 128
EF = 16
EH = 64
STEPS = 3

MSG_TILE = 1000


def _bn(x, g, b):
    m = jnp.mean(x, axis=0)
    v = jnp.var(x, axis=0)
    return g * (x - m) / jnp.sqrt(v + 1e-5) + b


def _msg_body(a_ref, e4_ref, w4t_ref, b4_ref, s_ref, out_ref):
    wt = jnp.dot(e4_ref[...], w4t_ref[...], preferred_element_type=jnp.float32)
    wt += b4_ref[0:1, :]
    a_rep = jnp.dot(a_ref[...], s_ref[...], preferred_element_type=jnp.float32)
    r = a_rep * wt
    r = r[:, :512] + r[:, 512:]
    r = r[:, :256] + r[:, 256:]
    r = r[:, :128] + r[:, 128:]
    r = r[:, :64] + r[:, 64:]
    out_ref[...] = r[:, :32] + r[:, 32:]


@functools.partial(jax.jit, static_argnames=("tile",))
def _msg_matmul(a_src, e4, w4t, b4row, sel, tile=MSG_TILE):
    return pl.pallas_call(
        _msg_body,
        grid=(E // tile,),
        in_specs=[
            pl.BlockSpec((tile, H), lambda i: (i, 0)),
            pl.BlockSpec((tile, EH), lambda i: (i, 0)),
            pl.BlockSpec((EH, H * H), lambda i: (0, 0)),
            pl.BlockSpec((8, H * H), lambda i: (0, 0)),
            pl.BlockSpec((H, H * H), lambda i: (0, 0)),
        ],
        out_specs=pl.BlockSpec((tile, H), lambda i: (i, 0)),
        out_shape=jax.ShapeDtypeStruct((E, H), jnp.float32),
    )(a_src, e4, w4t, b4row, sel)


_SC_NC = 2
_SC_NS = 16
_CH = 128
_NCHUNK = E // _CH
_CPC = _NCHUNK // _SC_NC
_STRIPE = 624
_TAIL = N - _STRIPE * _SC_NS


@jax.jit
def _sc_scatter_add(msg3, dst_flat, zstripe):
    mesh = plsc.VectorSubcoreMesh(core_axis_name="c", subcore_axis_name="s")

    @functools.partial(
        pl.kernel, mesh=mesh,
        out_type=jax.ShapeDtypeStruct((_SC_NC, N, H), jnp.float32),
        scratch_types=[
            pltpu.VMEM((_CH,), jnp.int32),
            pltpu.VMEM((_CH, H), jnp.float32),
            pltpu.VMEM_SHARED((N, H), jnp.float32),
        ],
        compiler_params=pltpu.CompilerParams(use_tc_tiling_on_sc=False),
    )
    def k(msg_hbm, dst_hbm, z_hbm, out_hbm, idx_v, rows_v, agg_sh):
        c = lax.axis_index("c")
        s = lax.axis_index("s")
        pltpu.sync_copy(z_hbm, agg_sh.at[pl.ds(s * _STRIPE, _STRIPE)])

        @pl.when(s == _SC_NS - 1)
        def _():
            pltpu.sync_copy(z_hbm.at[pl.ds(0, _TAIL)],
                            agg_sh.at[pl.ds(_STRIPE * _SC_NS, _TAIL)])
        plsc.subcore_barrier()

        def body(j, carry):
            cid_local = j * _SC_NS + s

            @pl.when(cid_local < _CPC)
            def _():
                cid = c * _CPC + cid_local
                pltpu.sync_copy(dst_hbm.at[pl.ds(cid * _CH, _CH)], idx_v)
                pltpu.sync_copy(msg_hbm.at[cid], rows_v)
                pltpu.sync_copy(rows_v, agg_sh.at[idx_v], add=True)
            return carry

        lax.fori_loop(0, (_CPC + _SC_NS - 1) // _SC_NS, body, 0)
        plsc.subcore_barrier()
        pltpu.sync_copy(agg_sh.at[pl.ds(s * _STRIPE, _STRIPE)],
                        out_hbm.at[c].at[pl.ds(s * _STRIPE, _STRIPE)])

        @pl.when(s == _SC_NS - 1)
        def _():
            pltpu.sync_copy(agg_sh.at[pl.ds(_STRIPE * _SC_NS, _TAIL)],
                            out_hbm.at[c].at[pl.ds(_STRIPE * _SC_NS, _TAIL)])

    return k(msg3, dst_flat, zstripe)


def _segment_sum_sc(msg, dst_flat, zstripe):
    parts = _sc_scatter_add(msg.reshape(_NCHUNK, _CH, H), dst_flat, zstripe)
    return parts[0] + parts[1]


def kernel(node, edge, edge_index, node_batch_index, coupling_index, coupling_type, coupling_type_back, coupling_value, coupling_batch_index, params):
    p = params
    ei = edge_index.T
    src = ei[0]
    dst = ei[1]
    x = jax.nn.relu(_bn(node, p['emb_bn1_g'], p['emb_bn1_b']) @ p['emb_W1'].T + p['emb_b1'])
    x = _bn(x, p['emb_bn2_g'], p['emb_bn2_b']) @ p['emb_W2'].T
    x = jax.nn.relu(x)
    h = x
    acc = x
    e = jax.nn.relu(_bn(edge, p['en_bn1_g'], p['en_bn1_b']) @ p['en_W1'].T + p['en_b1'])
    e = jax.nn.relu(_bn(e, p['en_bn2_g'], p['en_bn2_b']) @ p['en_W2'].T + p['en_b2'])
    e = jax.nn.relu(_bn(e, p['en_bn3_g'], p['en_bn3_b']) @ p['en_W3'].T + p['en_b3'])
    e4 = _bn(e, p['en_bn4_g'], p['en_bn4_b'])
    counts = jnp.maximum(jax.ops.segment_sum(jnp.ones(E, jnp.float32), dst, num_segments=N), 1.0)[:, None]
    w4t = p['en_W4'].T
    b4row = jnp.broadcast_to(p['en_b4'][None, :], (8, H * H))
    sel = jnp.repeat(jnp.eye(H, dtype=jnp.float32), H, axis=1)
    dst_flat = dst.astype(jnp.int32)
    zstripe = jnp.zeros((_STRIPE, H), jnp.float32)
    for _ in range(STEPS):
        a_src = acc[src]
        msg = _msg_matmul(a_src, e4, w4t, b4row, sel)
        agg = _segment_sum_sc(msg, dst_flat, zstripe) / counts
        m = jax.nn.relu(agg + acc @ p['conv_root'].T + p['conv_bias'])
        gi = m @ p['gru_W_ih'].T + p['gru_b_ih']
        gh = h @ p['gru_W_hh'].T + p['gru_b_hh']
        i_r, i_z, i_n = jnp.split(gi, 3, axis=1)
        h_r, h_z, h_n = jnp.split(gh, 3, axis=1)
        r = jax.nn.sigmoid(i_r + h_r)
        z = jax.nn.sigmoid(i_z + h_z)
        n = jnp.tanh(i_n + r * h_n)
        acc = (1.0 - z) * n + z * h
        h = acc
    q_star = jnp.zeros((B, 2 * H), jnp.float32)
    hl = jnp.zeros((B, H), jnp.float32)
    cl = jnp.zeros((B, H), jnp.float32)
    for _ in range(STEPS):
        gates = q_star @ p['lstm_W_ih'].T + p['lstm_b_ih'] + hl @ p['lstm_W_hh'].T + p['lstm_b_hh']
        gi_, gf_, gg_, go_ = jnp.split(gates, 4, axis=1)
        cl = jax.nn.sigmoid(gf_) * cl + jax.nn.sigmoid(gi_) * jnp.tanh(gg_)
        hl = jax.nn.sigmoid(go_) * jnp.tanh(cl)
        eatt = jnp.sum(acc * hl[node_batch_index], axis=-1)
        emax = jax.ops.segment_max(eatt, node_batch_index, num_segments=B)
        a = jnp.exp(eatt - emax[node_batch_index])
        denom = jax.ops.segment_sum(a, node_batch_index, num_segments=B)
        a = a / (denom[node_batch_index] + 1e-16)
        r_ = jax.ops.segment_sum(a[:, None] * acc, node_batch_index, num_segments=B)
        q_star = jnp.concatenate([hl, r_], axis=1)
    pool = q_star[coupling_batch_index]
    nf = acc[coupling_index.reshape(-1)].reshape(C, -1)
    feats = jnp.concatenate([pool, nf, coupling_type.astype(jnp.float32)], axis=-1)
    zf = jax.nn.relu(_bn(feats, p['fc_bn1_g'], p['fc_bn1_b']) @ p['fc_W1'].T + p['fc_b1'])
    preds = _bn(zf, p['fc_bn2_g'], p['fc_bn2_b']) @ p['fc_W2'].T + p['fc_b2']
    pred = jnp.take_along_axis(preds, coupling_type_back[:, None], axis=1).reshape(-1)
    return pred
